# Initial kernel scaffold; baseline (speedup 1.0000x reference)
import jax
import jax.numpy as jnp
from jax.experimental import pallas as pl


def kernel(seq, pos, params):
    raise NotImplementedError("write your pallas kernel here")


# speedup vs baseline: 1.4508x; 1.4508x over previous
import functools

import jax
import jax.numpy as jnp
from jax import lax
from jax.experimental import pallas as pl

_INTERPRET = False

K_NEIGHBORS = 6
NHEAD = 4
D = 64
FF = 2048
T = 32
BS = 8
ROWS = BS * T
NEG = -1e30


def _ln(x, g, b):
    m = jnp.mean(x, axis=-1, keepdims=True)
    v = jnp.mean((x - m) ** 2, axis=-1, keepdims=True)
    return (x - m) * lax.rsqrt(v + 1e-5) * g + b


def _softmax(x):
    m = jnp.max(x, axis=-1, keepdims=True)
    e = jnp.exp(x - m)
    return e / jnp.sum(e, axis=-1, keepdims=True)


def _sigmoid(x):
    return 1.0 / (1.0 + jnp.exp(-x))


def _head_masks():
    lane = lax.broadcasted_iota(jnp.int32, (1, D), 1) // (D // NHEAD)
    return [(lane == h).astype(jnp.float32) for h in range(NHEAD)]


def _tf_body(x_ref, ew, eb,
             wi0, bi0, wo0, bo0, g10, b10, w10, f10, w20, f20, g20, b20,
             wi1, bi1, wo1, bo1, g11, b11,
             out_ref):
    x = x_ref[...] @ ew[...] + eb[...]
    hms = _head_masks()
    scale = 1.0 / jnp.sqrt(jnp.float32(D // NHEAD))

    qkv = x @ wi0[...] + bi0[...]
    q = qkv[:, 0:D]
    k = qkv[:, D:2 * D]
    v = qkv[:, 2 * D:3 * D]
    ri = lax.broadcasted_iota(jnp.int32, (ROWS, ROWS), 0) // T
    ci = lax.broadcasted_iota(jnp.int32, (ROWS, ROWS), 1) // T
    seqmask = ri == ci
    o = jnp.zeros((ROWS, D), jnp.float32)
    for hm in hms:
        logits = lax.dot_general(q * hm, k, (((1,), (1,)), ((), ()))) * scale
        logits = jnp.where(seqmask, logits, NEG)
        o = o + _softmax(logits) @ (v * hm)
    x = _ln(x + (o @ wo0[...] + bo0[...]), g10[...], b10[...])
    f = jnp.maximum(x @ w10[...] + f10[...], 0.0) @ w20[...] + f20[...]
    x = _ln(x + f, g20[...], b20[...])

    qkv = x @ wi1[...] + bi1[...]
    q = qkv[:, 0:D]
    k = qkv[:, D:2 * D]
    v = qkv[:, 2 * D:3 * D]
    ri8 = lax.broadcasted_iota(jnp.int32, (BS, ROWS), 0)
    cj = lax.broadcasted_iota(jnp.int32, (BS, ROWS), 1)
    sel = (cj == T * ri8 + (T - 1)).astype(jnp.float32)
    x_last = sel @ x
    q_last = sel @ q
    amask = (cj // T) == ri8
    o = jnp.zeros((BS, D), jnp.float32)
    for hm in hms:
        logits = lax.dot_general(q_last * hm, k, (((1,), (1,)), ((), ()))) * scale
        logits = jnp.where(amask, logits, NEG)
        o = o + _softmax(logits) @ (v * hm)
    out_ref[...] = _ln(x_last + (o @ wo1[...] + bo1[...]), g11[...], b11[...])


def _head_body(y_ref, pos_ref, post_ref,
               w11, f11, w21, f21, g21, b21,
               qw, qb, kw, kb, vw, vb,
               mw1, mb1, mw2, mb2, hw, hb,
               hd1, hdb1, hd2, hdb2, mnw, mnb, vlw, vlb,
               mean_ref, val_ref, hearts_ref):
    y = y_ref[...]
    f = jnp.maximum(y @ w11[...] + f11[...], 0.0) @ w21[...] + f21[...]
    h = _ln(y + f, g21[...], b21[...])

    p = pos_ref[...]
    pt = post_ref[0]
    dx = p[:, 0:1] - pt[0:1, :]
    dy = p[:, 1:2] - pt[1:2, :]
    dist = jnp.sqrt(jnp.maximum(dx * dx + dy * dy, 0.0))
    ci = lax.broadcasted_iota(jnp.int32, dist.shape, 1)
    selmask = jnp.zeros(dist.shape, jnp.bool_)
    d = dist
    for _ in range(K_NEIGHBORS):
        vmin = jnp.min(d, axis=-1, keepdims=True)
        idx = jnp.min(jnp.where(d == vmin, ci, jnp.int32(1 << 30)),
                      axis=-1, keepdims=True)
        one = ci == idx
        selmask = selmask | one
        d = jnp.where(one, jnp.float32(3e38), d)

    q = h @ qw[...] + qb[...]
    km = h @ kw[...] + kb[...]
    vm = h @ vw[...] + vb[...]
    logits = lax.dot_general(q, km, (((1,), (1,)), ((), ()))) * (1.0 / jnp.sqrt(jnp.float32(D)))
    logits = jnp.where(selmask, logits, NEG)
    c = _softmax(logits) @ vm

    x2 = jnp.concatenate([h, c], axis=1)
    mcp = _sigmoid(jnp.maximum(h @ mw1[...] + mb1[...], 0.0) @ mw2[...] + mb2[...])
    hearts = _sigmoid(h @ hw[...] + hb[...])
    x2 = x2 * mcp
    t1 = jnp.maximum(x2 @ hd1[...] + hdb1[...], 0.0)
    t2 = jnp.maximum(t1 @ hd2[...] + hdb2[...], 0.0)
    mean_ref[0] = t2 @ mnw[...] + mnb[...]
    vv = t2 @ vlw[...] + vlb[...]
    val_ref[...] = jnp.broadcast_to(jnp.sum(vv) * (1.0 / 128.0), (1, 1, 128))
    hearts_ref[0] = hearts


@functools.partial(jax.jit, static_argnames=())
def kernel(seq, pos, params):
    B, Tn, N, F = seq.shape
    S = B * N
    x2d = seq.reshape(S * Tn, F)
    p = params

    def tw(name):
        return p[name].T

    def bw(name):
        return p[name].reshape(1, -1)

    full = lambda shape: pl.BlockSpec(shape, lambda i: (0,) * len(shape))

    a_ins = [
        x2d, tw('embed_W'), bw('embed_b'),
        tw('l0_inW'), bw('l0_inb'), tw('l0_outW'), bw('l0_outb'),
        bw('l0_ln1g'), bw('l0_ln1b'),
        tw('l0_W1'), bw('l0_b1'), tw('l0_W2'), bw('l0_b2'),
        bw('l0_ln2g'), bw('l0_ln2b'),
        tw('l1_inW'), bw('l1_inb'), tw('l1_outW'), bw('l1_outb'),
        bw('l1_ln1g'), bw('l1_ln1b'),
    ]
    a_specs = [pl.BlockSpec((ROWS, F), lambda i: (i, 0))] + \
              [full(a.shape) for a in a_ins[1:]]
    y = pl.pallas_call(
        _tf_body,
        grid=(S // BS,),
        in_specs=a_specs,
        out_specs=pl.BlockSpec((BS, D), lambda i: (i, 0)),
        out_shape=jax.ShapeDtypeStruct((S, D), jnp.float32),
        interpret=_INTERPRET,
    )(*a_ins)

    post = pos.transpose(0, 2, 1)
    heartsW = jnp.concatenate([p[f'heart_{n}_W'] for n in
                               ['black_soft', 'black_hard', 'black_crit', 'red', 'green']],
                              axis=0)
    heartsb = jnp.concatenate([p[f'heart_{n}_b'] for n in
                               ['black_soft', 'black_hard', 'black_crit', 'red', 'green']],
                              axis=0).reshape(1, 5)

    b_ins = [
        y, pos.reshape(B * N, 2), post,
        tw('l1_W1'), bw('l1_b1'), tw('l1_W2'), bw('l1_b2'),
        bw('l1_ln2g'), bw('l1_ln2b'),
        tw('q_W'), bw('q_b'), tw('k_W'), bw('k_b'), tw('v_W'), bw('v_b'),
        tw('mcp_W1'), bw('mcp_b1'), tw('mcp_W2'), bw('mcp_b2'),
        heartsW.T, heartsb,
        tw('head_W1'), bw('head_b1'), tw('head_W2'), bw('head_b2'),
        tw('mean_W'), bw('mean_b'), tw('value_W'), bw('value_b'),
    ]
    b_specs = [pl.BlockSpec((N, D), lambda i: (i, 0)),
               pl.BlockSpec((N, 2), lambda i: (i, 0)),
               pl.BlockSpec((1, 2, N), lambda i: (i, 0, 0))] + \
              [full(a.shape) for a in b_ins[3:]]
    mean3, val, hearts3 = pl.pallas_call(
        _head_body,
        grid=(B,),
        in_specs=b_specs,
        out_specs=[pl.BlockSpec((1, N, 2), lambda i: (i, 0, 0)),
                   pl.BlockSpec((1, 1, N), lambda i: (i, 0, 0)),
                   pl.BlockSpec((1, N, 5), lambda i: (i, 0, 0))],
        out_shape=[jax.ShapeDtypeStruct((B, N, 2), jnp.float32),
                   jax.ShapeDtypeStruct((B, 1, N), jnp.float32),
                   jax.ShapeDtypeStruct((B, N, 5), jnp.float32)],
        interpret=_INTERPRET,
    )(*b_ins)

    value = val[:, 0, 0:1]
    hearts = tuple(hearts3[:, :, i:i + 1] for i in range(5))
    return (mean3, value) + hearts
